# baseline (speedup 1.0000x reference)
import functools

import numpy as np

import jax
import jax.numpy as jnp
from jax import lax
from jax.experimental import pallas as pl
from jax.experimental.pallas import tpu as pltpu
from jax.experimental.pallas import tpu_sc as plsc

N, E, D, DE, H = 10000, 160000, 256, 16, 256
DH = D // 2
NS = 16
EPT = E // NS
K = 80
DW = DH // 2
NCHUNK = EPT // K
SUB0, SUB1 = 48, 32
SPC = 25
NST = NCHUNK // SPC
NP = 10240
PERM = np.array([32 * g + 2 * l + r
                 for g in range(4) for r in range(2) for l in range(16)],
                dtype=np.int32)
CPERM = np.concatenate([np.arange(128 * h + par, 128 * (h + 1), 2)
                        for h in range(2) for par in range(2)])
TPERM = np.concatenate([128 * h + PERM for h in range(2)])
RPT = NP // NS


def _bf16_bits(m):
    u = lax.bitcast_convert_type(m, jnp.int32)
    return (u + 0x7FFF + ((u >> 16) & 1)) >> 16


def _pack2(even, odd):
    return (_bf16_bits(even) & 0xFFFF) | (_bf16_bits(odd) << 16)


def _tc_ab_body(x_ref, wab_ref, t_ref):
    x = x_ref[...]
    a = jnp.dot(x, wab_ref[0], preferred_element_type=jnp.float32)
    b = jnp.dot(x, wab_ref[1], preferred_element_type=jnp.float32)
    t_ref[0] = a[:, :DH]
    t_ref[1] = a[:, DH:]
    t_ref[2] = b[:, :DH]
    t_ref[3] = b[:, DH:]


_tc_ab = pl.pallas_call(
    _tc_ab_body,
    grid=(5,),
    in_specs=[
        pl.BlockSpec((N // 5, D), lambda i: (i, 0)),
        pl.BlockSpec((2, D, D), lambda i: (0, 0, 0)),
    ],
    out_specs=pl.BlockSpec((4, N // 5, DH), lambda i: (0, i, 0)),
    out_shape=jax.ShapeDtypeStruct((4, N, DH), jnp.float32),
)

_EB = 2000


def _tc_c_body(ea_ref, wa_ref, be_ref, c_ref):
    cc = jnp.dot(ea_ref[...], wa_ref[...], preferred_element_type=jnp.float32)
    cc = cc + be_ref[...]
    c_ref[0] = _pack2(cc[:, 0:DW], cc[:, DW:2 * DW])
    c_ref[1] = _pack2(cc[:, 2 * DW:3 * DW], cc[:, 3 * DW:4 * DW])


_tc_c = pl.pallas_call(
    _tc_c_body,
    grid=(E // _EB,),
    in_specs=[
        pl.BlockSpec((_EB, DE), lambda i: (i, 0)),
        pl.BlockSpec((DE, D), lambda i: (0, 0)),
        pl.BlockSpec((1, D), lambda i: (0, 0)),
    ],
    out_specs=pl.BlockSpec((2, _EB, DW), lambda i: (0, i, 0)),
    out_shape=jax.ShapeDtypeStruct((2, E, DW), jnp.int32),
)


def _sc_body(t_hbm, c_hbm, ei_hbm, agg_hbm,
             sstage, dstage, isrc0, isrc1, idstt0, idstt1, idst0, idst1,
             ab0, ab1, bb0, bb1, cb0, cb1, agg_sh, semg, sems):
    c = lax.axis_index("c")
    s = lax.axis_index("s")

    def zrow0(i, carry):
        for j in range(DH // 16):
            ab0[i, pl.ds(16 * j, 16)] = jnp.zeros((16,), jnp.float32)
        return carry

    def zrow1(i, carry):
        for j in range(DH // 16):
            ab1[i, pl.ds(16 * j, 16)] = jnp.zeros((16,), jnp.float32)
        return carry

    lax.fori_loop(0, SUB0, zrow0, 0)
    lax.fori_loop(0, SUB1, zrow1, 0)
    for t in range(RPT // K):
        pltpu.sync_copy(ab0, agg_sh.at[pl.ds(s * RPT + t * K, SUB0)])
        pltpu.sync_copy(ab1, agg_sh.at[pl.ds(s * RPT + t * K + SUB0, SUB1)])
    plsc.subcore_barrier()

    mhi = jnp.int32(-65536)

    def unpack2(w):
        return (lax.bitcast_convert_type(w << 16, jnp.float32),
                lax.bitcast_convert_type(w & mhi, jnp.float32))

    off_a = c * N
    off_b = 2 * N + c * N

    def stage(st, carry0):
        sbase = s * EPT + st * (SPC * K)
        pltpu.sync_copy(ei_hbm.at[pl.ds(sbase, SPC * K)], sstage)
        pltpu.sync_copy(ei_hbm.at[pl.ds(E + sbase, SPC * K)], dstage)

        def chunk(kk, carry1):
            o = kk * K
            cbase = (c * (E // 2) + s * (EPT // 2)
                     + st * (SPC * K // 2) + kk * (K // 2))
            for j in range(K // 16):
                sj = pl.ds(16 * j, 16)
                sv = sstage[pl.ds(o + 16 * j, 16)] + off_a
                dv = dstage[pl.ds(o + 16 * j, 16)]
                if 16 * j < SUB0:
                    isrc0[sj] = sv
                    idst0[sj] = dv
                    idstt0[sj] = dv + off_b
                else:
                    sj1 = pl.ds(16 * j - SUB0, 16)
                    isrc1[sj1] = sv
                    idst1[sj1] = dv
                    idstt1[sj1] = dv + off_b
            base = sbase + o
            da0 = pltpu.async_copy(t_hbm.at[isrc0], ab0, semg)
            db0 = pltpu.async_copy(t_hbm.at[idstt0], bb0, semg)
            dc0 = pltpu.async_copy(
                c_hbm.at[pl.ds(cbase, SUB0 // 2)], cb0, semg)
            da1 = pltpu.async_copy(t_hbm.at[isrc1], ab1, semg)
            db1 = pltpu.async_copy(t_hbm.at[idstt1], bb1, semg)
            dc1 = pltpu.async_copy(
                c_hbm.at[pl.ds(cbase + SUB0 // 2, SUB1 // 2)], cb1, semg)

            def row0(i, carry):
                i2 = i // 2
                co = (i % 2) * DW
                for g in range(DW // 16):
                    clo, chi = unpack2(cb0[i2, pl.ds(co + 16 * g, 16)])
                    s0 = pl.ds(32 * g, 16)
                    s1 = pl.ds(32 * g + 16, 16)
                    ab0[i, s0] = jnp.maximum(ab0[i, s0] + bb0[i, s0] + clo,
                                             0.0)
                    ab0[i, s1] = jnp.maximum(ab0[i, s1] + bb0[i, s1] + chi,
                                             0.0)
                return carry

            def row1(i, carry):
                i2 = i // 2
                co = (i % 2) * DW
                for g in range(DW // 16):
                    clo, chi = unpack2(cb1[i2, pl.ds(co + 16 * g, 16)])
                    s0 = pl.ds(32 * g, 16)
                    s1 = pl.ds(32 * g + 16, 16)
                    ab1[i, s0] = jnp.maximum(ab1[i, s0] + bb1[i, s0] + clo,
                                             0.0)
                    ab1[i, s1] = jnp.maximum(ab1[i, s1] + bb1[i, s1] + chi,
                                             0.0)
                return carry

            da0.wait()
            db0.wait()
            dc0.wait()
            lax.fori_loop(0, SUB0, row0, 0)
            da1.wait()
            db1.wait()
            dc1.wait()
            ds0 = pltpu.async_copy(ab0, agg_sh.at[idst0], sems, add=True)
            lax.fori_loop(0, SUB1, row1, 0)
            ds0.wait()
            pltpu.sync_copy(ab1, agg_sh.at[idst1], add=True)
            return carry1

        lax.fori_loop(0, SPC, chunk, 0)
        return carry0

    lax.fori_loop(0, NST, stage, 0)
    plsc.subcore_barrier()

    pltpu.sync_copy(agg_sh.at[pl.ds(s * RPT, RPT)],
                    agg_hbm.at[pl.ds(c * NP + s * RPT, RPT)])


@functools.cache
def _make_sc_agg():
  return pl.kernel(
    _sc_body,
    out_type=jax.ShapeDtypeStruct((2 * NP, DH), jnp.float32),
    mesh=plsc.VectorSubcoreMesh(core_axis_name="c", subcore_axis_name="s"),
    scratch_types=[
        pltpu.VMEM((SPC * K,), jnp.int32),
        pltpu.VMEM((SPC * K,), jnp.int32),
        pltpu.VMEM((SUB0,), jnp.int32),
        pltpu.VMEM((SUB1,), jnp.int32),
        pltpu.VMEM((SUB0,), jnp.int32),
        pltpu.VMEM((SUB1,), jnp.int32),
        pltpu.VMEM((SUB0,), jnp.int32),
        pltpu.VMEM((SUB1,), jnp.int32),
        pltpu.VMEM((SUB0, DH), jnp.float32),
        pltpu.VMEM((SUB1, DH), jnp.float32),
        pltpu.VMEM((SUB0, DH), jnp.float32),
        pltpu.VMEM((SUB1, DH), jnp.float32),
        pltpu.VMEM((SUB0 // 2, 2 * DW), jnp.int32),
        pltpu.VMEM((SUB1 // 2, 2 * DW), jnp.int32),
        pltpu.VMEM_SHARED((NP, DH), jnp.float32),
        pltpu.SemaphoreType.DMA,
        pltpu.SemaphoreType.DMA,
    ],
  )


def _tc_fin_body(x_ref, agg_ref, wn_ref, bn_ref, w1_ref, b1_ref,
                 w2_ref, b2_ref, out_ref):
    h = jnp.dot(x_ref[...], wn_ref[0:D, :], preferred_element_type=jnp.float32)
    h = h + jnp.dot(agg_ref[0:N], wn_ref[D:D + DH, :],
                    preferred_element_type=jnp.float32)
    h = h + jnp.dot(agg_ref[NP:NP + N], wn_ref[D + DH:2 * D, :],
                    preferred_element_type=jnp.float32)
    h = jnp.maximum(h + bn_ref[...], 0.0)
    g = jnp.mean(h, axis=0, keepdims=True)
    z = jnp.maximum(jnp.dot(g, w1_ref[...], preferred_element_type=jnp.float32)
                    + b1_ref[...], 0.0)
    out_ref[...] = (jnp.dot(z, w2_ref[...], preferred_element_type=jnp.float32)
                    + b2_ref[...])


_tc_fin = pl.pallas_call(
    _tc_fin_body,
    out_shape=jax.ShapeDtypeStruct((1, 1), jnp.float32),
)


def kernel(x, edge_index, edge_attr, W_e, b_e, W_n, b_n, W1, b1, W2, b2):
    ei = edge_index.astype(jnp.int32).reshape(2 * E)
    wab = jnp.stack([W_e[:D][:, TPERM], W_e[D:2 * D][:, TPERM]])
    t4 = _tc_ab(x, wab)
    c2 = _tc_c(edge_attr, W_e[2 * D:][:, CPERM], b_e[CPERM].reshape(1, D))
    agg = _make_sc_agg()(t4.reshape(4 * N, DH), c2.reshape(E, 2 * DW), ei)
    wn_p = jnp.concatenate(
        [W_n[:D], W_n[D + PERM], W_n[D + DH + PERM]], axis=0)
    v = _tc_fin(x, agg, wn_p, b_n.reshape(1, D), W1, b1.reshape(1, H),
                W2, b2.reshape(1, 1))
    return v.reshape(1)


# speedup vs baseline: 1.8988x; 1.8988x over previous
import functools

import jax
import jax.numpy as jnp
from jax import lax
from jax.experimental import pallas as pl
from jax.experimental.pallas import tpu as pltpu
from jax.experimental.pallas import tpu_sc as plsc

N, E, D, DE, H = 10000, 160000, 256, 16, 256
DH = D // 2
NS = 16
EPT = E // NS
K = 80
NCHUNK = EPT // K
SUB0, SUB1 = 48, 32
SPC = 25
NST = NCHUNK // SPC
NP = 10240
RPT = NP // NS


def _tc_ab_body(x_ref, we_ref, t_ref):
    x = x_ref[...]
    a = jnp.dot(x, we_ref[0:D, :], preferred_element_type=jnp.float32)
    b = jnp.dot(x, we_ref[D:2 * D, :], preferred_element_type=jnp.float32)
    t_ref[0] = a[:, :DH]
    t_ref[1] = a[:, DH:]
    t_ref[2] = b[:, :DH]
    t_ref[3] = b[:, DH:]


_tc_ab = pl.pallas_call(
    _tc_ab_body,
    grid=(5,),
    in_specs=[
        pl.BlockSpec((N // 5, D), lambda i: (i, 0)),
        pl.BlockSpec((2 * D + DE, D), lambda i: (0, 0)),
    ],
    out_specs=pl.BlockSpec((4, N // 5, DH), lambda i: (0, i, 0)),
    out_shape=jax.ShapeDtypeStruct((4, N, DH), jnp.float32),
)

_EB = 2000


def _tc_c_body(ea_ref, wa_ref, be_ref, c_ref):
    cc = jnp.dot(ea_ref[...], wa_ref[...], preferred_element_type=jnp.float32)
    cc = cc + be_ref[...]
    c_ref[0] = cc[:, :DH]
    c_ref[1] = cc[:, DH:]


_tc_c = pl.pallas_call(
    _tc_c_body,
    grid=(E // _EB,),
    in_specs=[
        pl.BlockSpec((_EB, DE), lambda i: (i, 0)),
        pl.BlockSpec((DE, D), lambda i: (0, 0)),
        pl.BlockSpec((1, D), lambda i: (0, 0)),
    ],
    out_specs=pl.BlockSpec((2, _EB, DH), lambda i: (0, i, 0)),
    out_shape=jax.ShapeDtypeStruct((2, E, DH), jnp.float32),
)


def _sc_body(t_hbm, c_hbm, ei_hbm, agg_hbm,
             sstage, dstage, isrc0, isrc1, idstt0, idstt1, idst0, idst1,
             ab0, ab1, bb0, bb1, cb0, cb1, agg_sh, semg, sems):
    c = lax.axis_index("c")
    s = lax.axis_index("s")

    def zrow0(i, carry):
        for j in range(DH // 16):
            ab0[i, pl.ds(16 * j, 16)] = jnp.zeros((16,), jnp.float32)
        return carry

    def zrow1(i, carry):
        for j in range(DH // 16):
            ab1[i, pl.ds(16 * j, 16)] = jnp.zeros((16,), jnp.float32)
        return carry

    lax.fori_loop(0, SUB0, zrow0, 0)
    lax.fori_loop(0, SUB1, zrow1, 0)
    for t in range(RPT // K):
        pltpu.sync_copy(ab0, agg_sh.at[pl.ds(s * RPT + t * K, SUB0)])
        pltpu.sync_copy(ab1, agg_sh.at[pl.ds(s * RPT + t * K + SUB0, SUB1)])
    plsc.subcore_barrier()

    off_a = c * N
    off_b = 2 * N + c * N

    def stage(st, carry0):
        sbase = s * EPT + st * (SPC * K)
        pltpu.sync_copy(ei_hbm.at[pl.ds(sbase, SPC * K)], sstage)
        pltpu.sync_copy(ei_hbm.at[pl.ds(E + sbase, SPC * K)], dstage)

        def chunk(kk, carry1):
            o = kk * K
            for j in range(K // 16):
                sj = pl.ds(16 * j, 16)
                sv = sstage[pl.ds(o + 16 * j, 16)] + off_a
                dv = dstage[pl.ds(o + 16 * j, 16)]
                if 16 * j < SUB0:
                    isrc0[sj] = sv
                    idst0[sj] = dv
                    idstt0[sj] = dv + off_b
                else:
                    sj1 = pl.ds(16 * j - SUB0, 16)
                    isrc1[sj1] = sv
                    idst1[sj1] = dv
                    idstt1[sj1] = dv + off_b
            base = sbase + o
            da0 = pltpu.async_copy(t_hbm.at[isrc0], ab0, semg)
            db0 = pltpu.async_copy(t_hbm.at[idstt0], bb0, semg)
            dc0 = pltpu.async_copy(c_hbm.at[pl.ds(c * E + base, SUB0)],
                                   cb0, semg)
            da1 = pltpu.async_copy(t_hbm.at[isrc1], ab1, semg)
            db1 = pltpu.async_copy(t_hbm.at[idstt1], bb1, semg)
            dc1 = pltpu.async_copy(c_hbm.at[pl.ds(c * E + base + SUB0, SUB1)],
                                   cb1, semg)

            def row0(i, carry):
                for j in range(DH // 16):
                    sj = pl.ds(16 * j, 16)
                    v = ab0[i, sj] + bb0[i, sj] + cb0[i, sj]
                    ab0[i, sj] = jnp.maximum(v, 0.0)
                return carry

            def row1(i, carry):
                for j in range(DH // 16):
                    sj = pl.ds(16 * j, 16)
                    v = ab1[i, sj] + bb1[i, sj] + cb1[i, sj]
                    ab1[i, sj] = jnp.maximum(v, 0.0)
                return carry

            da0.wait()
            db0.wait()
            dc0.wait()
            lax.fori_loop(0, SUB0, row0, 0)
            da1.wait()
            db1.wait()
            dc1.wait()
            ds0 = pltpu.async_copy(ab0, agg_sh.at[idst0], sems, add=True)
            lax.fori_loop(0, SUB1, row1, 0)
            ds0.wait()
            pltpu.sync_copy(ab1, agg_sh.at[idst1], add=True)
            return carry1

        lax.fori_loop(0, SPC, chunk, 0)
        return carry0

    lax.fori_loop(0, NST, stage, 0)
    plsc.subcore_barrier()

    pltpu.sync_copy(agg_sh.at[pl.ds(s * RPT, RPT)],
                    agg_hbm.at[pl.ds(c * NP + s * RPT, RPT)])


@functools.cache
def _make_sc_agg():
  return pl.kernel(
    _sc_body,
    out_type=jax.ShapeDtypeStruct((2 * NP, DH), jnp.float32),
    mesh=plsc.VectorSubcoreMesh(core_axis_name="c", subcore_axis_name="s"),
    scratch_types=[
        pltpu.VMEM((SPC * K,), jnp.int32),
        pltpu.VMEM((SPC * K,), jnp.int32),
        pltpu.VMEM((SUB0,), jnp.int32),
        pltpu.VMEM((SUB1,), jnp.int32),
        pltpu.VMEM((SUB0,), jnp.int32),
        pltpu.VMEM((SUB1,), jnp.int32),
        pltpu.VMEM((SUB0,), jnp.int32),
        pltpu.VMEM((SUB1,), jnp.int32),
        pltpu.VMEM((SUB0, DH), jnp.float32),
        pltpu.VMEM((SUB1, DH), jnp.float32),
        pltpu.VMEM((SUB0, DH), jnp.float32),
        pltpu.VMEM((SUB1, DH), jnp.float32),
        pltpu.VMEM((SUB0, DH), jnp.float32),
        pltpu.VMEM((SUB1, DH), jnp.float32),
        pltpu.VMEM_SHARED((NP, DH), jnp.float32),
        pltpu.SemaphoreType.DMA,
        pltpu.SemaphoreType.DMA,
    ],
  )


def _tc_fin_body(x_ref, agg_ref, wn_ref, bn_ref, w1_ref, b1_ref,
                 w2_ref, b2_ref, out_ref):
    h = jnp.dot(x_ref[...], wn_ref[0:D, :], preferred_element_type=jnp.float32)
    h = h + jnp.dot(agg_ref[0:N], wn_ref[D:D + DH, :],
                    preferred_element_type=jnp.float32)
    h = h + jnp.dot(agg_ref[NP:NP + N], wn_ref[D + DH:2 * D, :],
                    preferred_element_type=jnp.float32)
    h = jnp.maximum(h + bn_ref[...], 0.0)
    g = jnp.mean(h, axis=0, keepdims=True)
    z = jnp.maximum(jnp.dot(g, w1_ref[...], preferred_element_type=jnp.float32)
                    + b1_ref[...], 0.0)
    out_ref[...] = (jnp.dot(z, w2_ref[...], preferred_element_type=jnp.float32)
                    + b2_ref[...])


_tc_fin = pl.pallas_call(
    _tc_fin_body,
    out_shape=jax.ShapeDtypeStruct((1, 1), jnp.float32),
)


def kernel(x, edge_index, edge_attr, W_e, b_e, W_n, b_n, W1, b1, W2, b2):
    ei = edge_index.astype(jnp.int32).reshape(2 * E)
    t4 = _tc_ab(x, W_e)
    c2 = _tc_c(edge_attr, W_e[2 * D:], b_e.reshape(1, D))
    agg = _make_sc_agg()(t4.reshape(4 * N, DH), c2.reshape(2 * E, DH), ei)
    v = _tc_fin(x, agg, W_n, b_n.reshape(1, D), W1, b1.reshape(1, H),
                W2, b2.reshape(1, 1))
    return v.reshape(1)
